# Initial kernel scaffold; baseline (speedup 1.0000x reference)
import jax
import jax.numpy as jnp
from jax.experimental import pallas as pl


def kernel(x, expert_w, gate_w, gate_b):
    raise NotImplementedError("write your pallas kernel here")


# speedup vs baseline: 1.1915x; 1.1915x over previous
import functools

import jax
import jax.numpy as jnp
from jax.experimental import pallas as pl

_B, _CI, _H, _W = 2, 96, 224, 224
_CO = 192
_E = 8
_HO = _WO = 112
_KTAB = 3 * 3 * _CI * _CO
_PREC = jax.lax.Precision.HIGHEST


def _pool_kernel(x_ref, o_ref):
    h = pl.program_id(1)

    @pl.when(h == 0)
    def _init():
        o_ref[0] = jnp.zeros_like(o_ref[0])

    o_ref[0] += jnp.sum(x_ref[0], axis=(0, 1), keepdims=True)[0]


def _gate_kernel(p_ref, gw_ref, gb_ref, wt_ref, ew_ref, wc_ref):
    pooled = p_ref[:, 0, :] * (1.0 / (_H * _W))
    logits = jnp.dot(pooled, gw_ref[...], precision=_PREC) + gb_ref[...]
    iota = jax.lax.broadcasted_iota(jnp.int32, (_B, _E), 1)
    m1 = jnp.max(logits, axis=1, keepdims=True)
    i1 = jnp.min(jnp.where(logits == m1, iota, _E), axis=1, keepdims=True)
    masked = jnp.where(iota == i1, -jnp.inf, logits)
    m2 = jnp.max(masked, axis=1, keepdims=True)
    i2 = jnp.min(jnp.where(masked == m2, iota, _E), axis=1, keepdims=True)
    d = jnp.exp(m2 - m1)
    g1 = 1.0 / (1.0 + d)
    ew = jnp.where(iota == i1, g1, 0.0) + jnp.where(iota == i2, g1 * d, 0.0)
    ew_ref[...] = ew
    for b in range(_B):
        acc = jnp.zeros((1, _KTAB), dtype=jnp.float32)
        for e in range(_E):
            acc = acc + wt_ref[e : e + 1, :] * ew[b : b + 1, e : e + 1]
        wc_ref[b : b + 1, :] = acc


def _conv_kernel(a0_ref, a1_ref, w_ref, o_ref, *, bc):
    acc = None
    for dy in range(3):
        rs = 1 if dy == 1 else 0
        rpo = 1 if dy == 2 else 0
        x0 = a0_ref[0, rpo : rpo + _HO, rs, :, :]
        x1 = a1_ref[0, rpo : rpo + _HO, rs, :, 0:_CI]
        t = jnp.dot(x0.reshape(_HO * bc, 2 * _CI),
                    w_ref[0, dy, 0 : 2 * _CI, :], precision=_PREC)
        t = t + jnp.dot(x1.reshape(_HO * bc, _CI),
                        w_ref[0, dy, 2 * _CI : 3 * _CI, :], precision=_PREC)
        acc = t if acc is None else acc + t
    o_ref[0] = acc.reshape(_HO, bc, _CO)


def kernel(x, expert_w, gate_w, gate_b):
    x_t = jnp.transpose(x, (0, 2, 3, 1))
    xp = jnp.pad(x_t, ((0, 0), (1, 1), (1, 1), (0, 0)))
    a = xp.reshape(_B, 113, 2, 113, 2 * _CI)
    a1 = a[:, :, :, 1:, :]
    wt = jnp.transpose(expert_w, (0, 3, 4, 2, 1)).reshape(_E, _KTAB)

    hb = 28
    pooled = pl.pallas_call(
        _pool_kernel,
        grid=(_B, _H // hb),
        in_specs=[pl.BlockSpec((1, hb, _W, _CI), lambda b, h: (b, h, 0, 0))],
        out_specs=pl.BlockSpec((1, 1, _CI), lambda b, h: (b, 0, 0)),
        out_shape=jax.ShapeDtypeStruct((_B, 1, _CI), jnp.float32),
    )(x_t)

    ew, wc_flat = pl.pallas_call(
        _gate_kernel,
        in_specs=[
            pl.BlockSpec((_B, 1, _CI), lambda: (0, 0, 0)),
            pl.BlockSpec((_CI, _E), lambda: (0, 0)),
            pl.BlockSpec((1, _E), lambda: (0, 0)),
            pl.BlockSpec((_E, _KTAB), lambda: (0, 0)),
        ],
        out_specs=[
            pl.BlockSpec((_B, _E), lambda: (0, 0)),
            pl.BlockSpec((_B, _KTAB), lambda: (0, 0)),
        ],
        out_shape=[
            jax.ShapeDtypeStruct((_B, _E), jnp.float32),
            jax.ShapeDtypeStruct((_B, _KTAB), jnp.float32),
        ],
    )(pooled, gate_w, gate_b.reshape(1, _E), wt)
    wc = wc_flat.reshape(_B, 3, 3 * _CI, _CO)

    bc = 16
    out_t = pl.pallas_call(
        functools.partial(_conv_kernel, bc=bc),
        grid=(_B, _WO // bc),
        in_specs=[
            pl.BlockSpec((1, 113, 2, bc, 2 * _CI), lambda b, c: (b, 0, 0, c, 0)),
            pl.BlockSpec((1, 113, 2, bc, 2 * _CI), lambda b, c: (b, 0, 0, c, 0)),
            pl.BlockSpec((1, 3, 3 * _CI, _CO), lambda b, c: (b, 0, 0, 0)),
        ],
        out_specs=pl.BlockSpec((1, _HO, bc, _CO), lambda b, c: (b, 0, c, 0)),
        out_shape=jax.ShapeDtypeStruct((_B, _HO, _WO, _CO), jnp.float32),
    )(a, a1, wc)

    out = jnp.transpose(out_t, (0, 3, 1, 2))
    return (out, ew)


# speedup vs baseline: 1.9538x; 1.6397x over previous
import functools

import jax
import jax.numpy as jnp
from jax.experimental import pallas as pl

_B, _CI, _H, _W = 2, 96, 224, 224
_CO = 192
_E = 8
_HO = _WO = 112
_KTAB = 3 * 3 * _CI * _CO
_PREC = jax.lax.Precision.HIGHEST


def _pool_kernel(x_ref, o_ref):
    h = pl.program_id(1)

    @pl.when(h == 0)
    def _init():
        o_ref[0] = jnp.zeros_like(o_ref[0])

    o_ref[0] += jnp.sum(x_ref[0], axis=1, keepdims=True)


def _gate_kernel(p_ref, gw_ref, gb_ref, wt_ref, ew_ref, wc_ref):
    pooled = p_ref[:, 0, :] * (1.0 / (_H * _W))
    logits = jnp.dot(pooled, gw_ref[...], precision=_PREC) + gb_ref[...]
    iota = jax.lax.broadcasted_iota(jnp.int32, (_B, _E), 1)
    m1 = jnp.max(logits, axis=1, keepdims=True)
    i1 = jnp.min(jnp.where(logits == m1, iota, _E), axis=1, keepdims=True)
    masked = jnp.where(iota == i1, -jnp.inf, logits)
    m2 = jnp.max(masked, axis=1, keepdims=True)
    i2 = jnp.min(jnp.where(masked == m2, iota, _E), axis=1, keepdims=True)
    d = jnp.exp(m2 - m1)
    g1 = 1.0 / (1.0 + d)
    ew = jnp.where(iota == i1, g1, 0.0) + jnp.where(iota == i2, g1 * d, 0.0)
    ew_ref[...] = ew
    for b in range(_B):
        acc = jnp.zeros((1, _KTAB), dtype=jnp.float32)
        for e in range(_E):
            acc = acc + wt_ref[e : e + 1, :] * ew[b : b + 1, e : e + 1]
        wc_ref[b : b + 1, :] = acc.astype(jnp.bfloat16)


def _conv_kernel(a0_ref, a1_ref, w_ref, o_ref, *, bc):
    acc = None
    for dy in range(3):
        rs = 1 if dy == 1 else 0
        rpo = 1 if dy == 2 else 0
        x0 = a0_ref[0, rpo : rpo + _HO, rs, :, :]
        x1 = a1_ref[0, rpo : rpo + _HO, rs, :, 0:_CI]
        t = jnp.dot(x0.reshape(_HO * bc, 2 * _CI),
                    w_ref[0, dy, 0 : 2 * _CI, :],
                    preferred_element_type=jnp.float32)
        t = t + jnp.dot(x1.reshape(_HO * bc, _CI),
                        w_ref[0, dy, 2 * _CI : 3 * _CI, :],
                        preferred_element_type=jnp.float32)
        acc = t if acc is None else acc + t
    o_ref[0] = acc.reshape(_HO, bc, _CO)


def kernel(x, expert_w, gate_w, gate_b):
    x_t = jnp.transpose(x, (0, 2, 3, 1))
    xp = jnp.pad(x_t.astype(jnp.bfloat16),
                 ((0, 0), (1, 1), (1, 1), (0, 0)))
    a = xp.reshape(_B, 113, 2, 113, 2 * _CI)
    a1 = a[:, :, :, 1:, :]
    wt = jnp.transpose(expert_w, (0, 3, 4, 2, 1)).reshape(_E, _KTAB)

    lb = (_H * _W) // 8
    pooled_c1 = pl.pallas_call(
        _pool_kernel,
        grid=(_B, 8),
        in_specs=[pl.BlockSpec((1, _CI, lb), lambda b, h: (b, 0, h))],
        out_specs=pl.BlockSpec((1, _CI, 1), lambda b, h: (b, 0, 0)),
        out_shape=jax.ShapeDtypeStruct((_B, _CI, 1), jnp.float32),
    )(x.reshape(_B, _CI, _H * _W))
    pooled = jnp.transpose(pooled_c1, (0, 2, 1))

    ew, wc_flat = pl.pallas_call(
        _gate_kernel,
        in_specs=[
            pl.BlockSpec((_B, 1, _CI), lambda: (0, 0, 0)),
            pl.BlockSpec((_CI, _E), lambda: (0, 0)),
            pl.BlockSpec((1, _E), lambda: (0, 0)),
            pl.BlockSpec((_E, _KTAB), lambda: (0, 0)),
        ],
        out_specs=[
            pl.BlockSpec((_B, _E), lambda: (0, 0)),
            pl.BlockSpec((_B, _KTAB), lambda: (0, 0)),
        ],
        out_shape=[
            jax.ShapeDtypeStruct((_B, _E), jnp.float32),
            jax.ShapeDtypeStruct((_B, _KTAB), jnp.bfloat16),
        ],
    )(pooled, gate_w, gate_b.reshape(1, _E), wt)
    wc = wc_flat.reshape(_B, 3, 3 * _CI, _CO)

    bc = 16
    out_t = pl.pallas_call(
        functools.partial(_conv_kernel, bc=bc),
        grid=(_B, _WO // bc),
        in_specs=[
            pl.BlockSpec((1, 113, 2, bc, 2 * _CI), lambda b, c: (b, 0, 0, c, 0)),
            pl.BlockSpec((1, 113, 2, bc, 2 * _CI), lambda b, c: (b, 0, 0, c, 0)),
            pl.BlockSpec((1, 3, 3 * _CI, _CO), lambda b, c: (b, 0, 0, 0)),
        ],
        out_specs=pl.BlockSpec((1, _HO, bc, _CO), lambda b, c: (b, 0, c, 0)),
        out_shape=jax.ShapeDtypeStruct((_B, _HO, _WO, _CO), jnp.float32),
    )(a, a1, wc)

    out = jnp.transpose(out_t, (0, 3, 1, 2))
    return (out, ew)


# speedup vs baseline: 2.7194x; 1.3918x over previous
import functools

import jax
import jax.numpy as jnp
from jax.experimental import pallas as pl
from jax.experimental.pallas import tpu as pltpu

_B, _CI, _H, _W = 2, 96, 224, 224
_CO = 192
_E = 8
_HO = _WO = 112
_KTAB = 3 * 3 * _CI * _CO
_PREC = jax.lax.Precision.HIGHEST
_RB = 32
_NH = _H // _RB


def _repack_kernel(x_ref, s_ref, a_ref, a1_ref, p_ref, scr_ref):
    h = pl.program_id(1)
    first = jnp.where(h > 0, scr_ref[...], 0.0)
    body = jnp.where(h < _NH, x_ref[0, :, 0 : _RB - 1, :], 0.0)
    vf = jnp.concatenate([first, body], axis=1)

    @pl.when(h == 0)
    def _init():
        p_ref[0] = jnp.zeros_like(p_ref[0])

    p_ref[0] += jnp.sum(jnp.sum(vf, axis=2), axis=1, keepdims=True)
    scr_ref[...] = x_ref[0, :, _RB - 1 : _RB, :]

    vm = vf.astype(jnp.bfloat16).reshape(_CI * _RB, _W)
    m = jnp.dot(vm, s_ref[...],
                preferred_element_type=jnp.float32).astype(jnp.bfloat16)
    m3 = m.reshape(_CI, _RB, 256)
    for q in range(_RB):
        me = m3[:, q, 0:113]
        mo = m3[:, q, 128:241]
        c = jnp.concatenate([me, mo], axis=0)
        t = jnp.transpose(c, (1, 0))
        a_ref[0, q // 2, q % 2] = t
        a1_ref[0, q // 2, q % 2] = t[1:113, :]


def _gate_kernel(p_ref, gw_ref, gb_ref, wt_ref, ew_ref, wc_ref):
    pooled = p_ref[:, 0, :] * (1.0 / (_H * _W))
    logits = jnp.dot(pooled, gw_ref[...], precision=_PREC) + gb_ref[...]
    iota = jax.lax.broadcasted_iota(jnp.int32, (_B, _E), 1)
    m1 = jnp.max(logits, axis=1, keepdims=True)
    i1 = jnp.min(jnp.where(logits == m1, iota, _E), axis=1, keepdims=True)
    masked = jnp.where(iota == i1, -jnp.inf, logits)
    m2 = jnp.max(masked, axis=1, keepdims=True)
    i2 = jnp.min(jnp.where(masked == m2, iota, _E), axis=1, keepdims=True)
    d = jnp.exp(m2 - m1)
    g1 = 1.0 / (1.0 + d)
    ew = jnp.where(iota == i1, g1, 0.0) + jnp.where(iota == i2, g1 * d, 0.0)
    ew_ref[...] = ew
    for b in range(_B):
        acc = jnp.zeros((1, _KTAB), dtype=jnp.float32)
        for e in range(_E):
            acc = acc + wt_ref[e : e + 1, :] * ew[b : b + 1, e : e + 1]
        wc_ref[b : b + 1, :] = acc.astype(jnp.bfloat16)


def _conv_kernel(a0_ref, a1_ref, w_ref, o_ref, *, bc):
    acc = None
    for dy in range(3):
        rs = 1 if dy == 1 else 0
        rpo = 1 if dy == 2 else 0
        x0 = a0_ref[0, rpo : rpo + _HO, rs, :, :]
        x1 = a1_ref[0, rpo : rpo + _HO, rs, :, 0:_CI]
        t = jnp.dot(x0.reshape(_HO * bc, 2 * _CI),
                    w_ref[0, dy, 0 : 2 * _CI, :],
                    preferred_element_type=jnp.float32)
        t = t + jnp.dot(x1.reshape(_HO * bc, _CI),
                        w_ref[0, dy, 2 * _CI : 3 * _CI, :],
                        preferred_element_type=jnp.float32)
        acc = t if acc is None else acc + t
    o_ref[0] = acc.reshape(_HO, bc, _CO)


def _selection_matrix():
    c = jnp.arange(_W)[:, None]
    j = jnp.arange(256)[None, :]
    even = (j < 113) & (c == 2 * j - 1)
    odd = (j >= 128) & (j < 241) & (c == 2 * (j - 128))
    return (even | odd).astype(jnp.bfloat16)


def kernel(x, expert_w, gate_w, gate_b):
    sel = _selection_matrix()
    wt = (jnp.transpose(expert_w, (0, 3, 4, 2, 1))
          .reshape(_E, _KTAB).astype(jnp.bfloat16))

    a, a1, pooled_c1 = pl.pallas_call(
        _repack_kernel,
        grid=(_B, _NH + 1),
        in_specs=[
            pl.BlockSpec((1, _CI, _RB, _W),
                         lambda b, h: (b, 0, jnp.minimum(h, _NH - 1), 0)),
            pl.BlockSpec((_W, 256), lambda b, h: (0, 0)),
        ],
        out_specs=[
            pl.BlockSpec((1, _RB // 2, 2, 113, 2 * _CI),
                         lambda b, h: (b, h, 0, 0, 0)),
            pl.BlockSpec((1, _RB // 2, 2, _WO, 2 * _CI),
                         lambda b, h: (b, h, 0, 0, 0)),
            pl.BlockSpec((1, _CI, 1), lambda b, h: (b, 0, 0)),
        ],
        out_shape=[
            jax.ShapeDtypeStruct((_B, 113, 2, 113, 2 * _CI), jnp.bfloat16),
            jax.ShapeDtypeStruct((_B, 113, 2, _WO, 2 * _CI), jnp.bfloat16),
            jax.ShapeDtypeStruct((_B, _CI, 1), jnp.float32),
        ],
        scratch_shapes=[pltpu.VMEM((_CI, 1, _W), jnp.float32)],
    )(x, sel)
    pooled = jnp.transpose(pooled_c1, (0, 2, 1))

    ew, wc_flat = pl.pallas_call(
        _gate_kernel,
        in_specs=[
            pl.BlockSpec((_B, 1, _CI), lambda: (0, 0, 0)),
            pl.BlockSpec((_CI, _E), lambda: (0, 0)),
            pl.BlockSpec((1, _E), lambda: (0, 0)),
            pl.BlockSpec((_E, _KTAB), lambda: (0, 0)),
        ],
        out_specs=[
            pl.BlockSpec((_B, _E), lambda: (0, 0)),
            pl.BlockSpec((_B, _KTAB), lambda: (0, 0)),
        ],
        out_shape=[
            jax.ShapeDtypeStruct((_B, _E), jnp.float32),
            jax.ShapeDtypeStruct((_B, _KTAB), jnp.bfloat16),
        ],
    )(pooled, gate_w, gate_b.reshape(1, _E), wt)
    wc = wc_flat.reshape(_B, 3, 3 * _CI, _CO)

    bc = 16
    out_t = pl.pallas_call(
        functools.partial(_conv_kernel, bc=bc),
        grid=(_B, _WO // bc),
        in_specs=[
            pl.BlockSpec((1, 113, 2, bc, 2 * _CI), lambda b, c: (b, 0, 0, c, 0)),
            pl.BlockSpec((1, 113, 2, bc, 2 * _CI), lambda b, c: (b, 0, 0, c, 0)),
            pl.BlockSpec((1, 3, 3 * _CI, _CO), lambda b, c: (b, 0, 0, 0)),
        ],
        out_specs=pl.BlockSpec((1, _HO, bc, _CO), lambda b, c: (b, 0, c, 0)),
        out_shape=jax.ShapeDtypeStruct((_B, _HO, _WO, _CO), jnp.float32),
    )(a, a1, wc)

    out = jnp.transpose(out_t, (0, 3, 1, 2))
    return (out, ew)


# speedup vs baseline: 4.2871x; 1.5765x over previous
import functools

import jax
import jax.numpy as jnp
from jax.experimental import pallas as pl
from jax.experimental.pallas import tpu as pltpu

_B, _CI, _H, _W = 2, 96, 224, 224
_CO = 192
_E = 8
_HO = _WO = 112
_KTAB = 3 * 3 * _CI * _CO
_PREC = jax.lax.Precision.HIGHEST
_RB = 32
_NH = _H // _RB


def _repack_kernel(x_ref, s_ref, a_ref, a1_ref, p_ref, scr_ref):
    h = pl.program_id(1)
    first = jnp.where(h > 0, scr_ref[...], 0.0)
    body = jnp.where(h < _NH, x_ref[0, :, 0 : _RB - 1, :], 0.0)
    vf = jnp.concatenate([first, body], axis=1)

    @pl.when(h == 0)
    def _init():
        p_ref[0] = jnp.zeros_like(p_ref[0])

    p_ref[0] += jnp.sum(jnp.sum(vf, axis=2), axis=1, keepdims=True)
    scr_ref[...] = x_ref[0, :, _RB - 1 : _RB, :]

    vm = vf.astype(jnp.bfloat16).reshape(_CI * _RB, _W)
    m = jnp.dot(vm, s_ref[...],
                preferred_element_type=jnp.float32).astype(jnp.bfloat16)
    m3 = m.reshape(_CI, _RB, 256)
    for q in range(_RB):
        me = m3[:, q, 0:113]
        mo = m3[:, q, 128:241]
        c = jnp.concatenate([me, mo], axis=0)
        t = jnp.transpose(c, (1, 0))
        a_ref[0, q // 2, q % 2] = t
        a1_ref[0, q // 2, q % 2] = t[1:113, :]


def _nchw_kernel(i_ref, o_ref, *, rb):
    for r in range(rb):
        o_ref[0, :, r, :] = jnp.transpose(i_ref[0, r], (1, 0))


def _gate_kernel(p_ref, gw_ref, gb_ref, wt_ref, ew_ref, wc_ref):
    pooled = p_ref[:, 0, :] * (1.0 / (_H * _W))
    logits = jnp.dot(pooled, gw_ref[...], precision=_PREC) + gb_ref[...]
    iota = jax.lax.broadcasted_iota(jnp.int32, (_B, _E), 1)
    m1 = jnp.max(logits, axis=1, keepdims=True)
    i1 = jnp.min(jnp.where(logits == m1, iota, _E), axis=1, keepdims=True)
    masked = jnp.where(iota == i1, -jnp.inf, logits)
    m2 = jnp.max(masked, axis=1, keepdims=True)
    i2 = jnp.min(jnp.where(masked == m2, iota, _E), axis=1, keepdims=True)
    d = jnp.exp(m2 - m1)
    g1 = 1.0 / (1.0 + d)
    ew = jnp.where(iota == i1, g1, 0.0) + jnp.where(iota == i2, g1 * d, 0.0)
    ew_ref[...] = ew
    for b in range(_B):
        acc = jnp.zeros((1, _KTAB), dtype=jnp.float32)
        for e in range(_E):
            acc = acc + wt_ref[e : e + 1, :] * ew[b : b + 1, e : e + 1]
        wc_ref[b : b + 1, :] = acc.astype(jnp.bfloat16)


def _conv_kernel(a0_ref, a1_ref, w_ref, o_ref, *, bc):
    acc = None
    for dy in range(3):
        rs = 1 if dy == 1 else 0
        rpo = 1 if dy == 2 else 0
        x0 = a0_ref[0, rpo : rpo + _HO, rs, :, :]
        x1 = a1_ref[0, rpo : rpo + _HO, rs, :, 0:_CI]
        t = jnp.dot(x0.reshape(_HO * bc, 2 * _CI),
                    w_ref[0, dy, 0 : 2 * _CI, :],
                    preferred_element_type=jnp.float32)
        t = t + jnp.dot(x1.reshape(_HO * bc, _CI),
                        w_ref[0, dy, 2 * _CI : 3 * _CI, :],
                        preferred_element_type=jnp.float32)
        acc = t if acc is None else acc + t
    o_ref[0] = acc.reshape(_HO, bc, _CO)


def _selection_matrix():
    c = jnp.arange(_W)[:, None]
    j = jnp.arange(256)[None, :]
    even = (j < 113) & (c == 2 * j - 1)
    odd = (j >= 128) & (j < 241) & (c == 2 * (j - 128))
    return (even | odd).astype(jnp.bfloat16)


def kernel(x, expert_w, gate_w, gate_b):
    sel = _selection_matrix()
    wt = (jnp.transpose(expert_w, (0, 3, 4, 2, 1))
          .reshape(_E, _KTAB).astype(jnp.bfloat16))

    a, a1, pooled_c1 = pl.pallas_call(
        _repack_kernel,
        grid=(_B, _NH + 1),
        in_specs=[
            pl.BlockSpec((1, _CI, _RB, _W),
                         lambda b, h: (b, 0, jnp.minimum(h, _NH - 1), 0)),
            pl.BlockSpec((_W, 256), lambda b, h: (0, 0)),
        ],
        out_specs=[
            pl.BlockSpec((1, _RB // 2, 2, 113, 2 * _CI),
                         lambda b, h: (b, h, 0, 0, 0)),
            pl.BlockSpec((1, _RB // 2, 2, _WO, 2 * _CI),
                         lambda b, h: (b, h, 0, 0, 0)),
            pl.BlockSpec((1, _CI, 1), lambda b, h: (b, 0, 0)),
        ],
        out_shape=[
            jax.ShapeDtypeStruct((_B, 113, 2, 113, 2 * _CI), jnp.bfloat16),
            jax.ShapeDtypeStruct((_B, 113, 2, _WO, 2 * _CI), jnp.bfloat16),
            jax.ShapeDtypeStruct((_B, _CI, 1), jnp.float32),
        ],
        scratch_shapes=[pltpu.VMEM((_CI, 1, _W), jnp.float32)],
    )(x, sel)
    pooled = jnp.transpose(pooled_c1, (0, 2, 1))

    ew, wc_flat = pl.pallas_call(
        _gate_kernel,
        in_specs=[
            pl.BlockSpec((_B, 1, _CI), lambda: (0, 0, 0)),
            pl.BlockSpec((_CI, _E), lambda: (0, 0)),
            pl.BlockSpec((1, _E), lambda: (0, 0)),
            pl.BlockSpec((_E, _KTAB), lambda: (0, 0)),
        ],
        out_specs=[
            pl.BlockSpec((_B, _E), lambda: (0, 0)),
            pl.BlockSpec((_B, _KTAB), lambda: (0, 0)),
        ],
        out_shape=[
            jax.ShapeDtypeStruct((_B, _E), jnp.float32),
            jax.ShapeDtypeStruct((_B, _KTAB), jnp.bfloat16),
        ],
    )(pooled, gate_w, gate_b.reshape(1, _E), wt)
    wc = wc_flat.reshape(_B, 3, 3 * _CI, _CO)

    bc = 16
    out_t = pl.pallas_call(
        functools.partial(_conv_kernel, bc=bc),
        grid=(_B, _WO // bc),
        in_specs=[
            pl.BlockSpec((1, 113, 2, bc, 2 * _CI), lambda b, c: (b, 0, 0, c, 0)),
            pl.BlockSpec((1, 113, 2, bc, 2 * _CI), lambda b, c: (b, 0, 0, c, 0)),
            pl.BlockSpec((1, 3, 3 * _CI, _CO), lambda b, c: (b, 0, 0, 0)),
        ],
        out_specs=pl.BlockSpec((1, _HO, bc, _CO), lambda b, c: (b, 0, c, 0)),
        out_shape=jax.ShapeDtypeStruct((_B, _HO, _WO, _CO), jnp.float32),
    )(a, a1, wc)

    rb = 16
    out = pl.pallas_call(
        functools.partial(_nchw_kernel, rb=rb),
        grid=(_B, _HO // rb),
        in_specs=[pl.BlockSpec((1, rb, _WO, _CO), lambda b, r: (b, r, 0, 0))],
        out_specs=pl.BlockSpec((1, _CO, rb, _WO), lambda b, r: (b, 0, r, 0)),
        out_shape=jax.ShapeDtypeStruct((_B, _CO, _HO, _WO), jnp.float32),
    )(out_t)
    return (out, ew)
